# baseline (device time: 101124 ns/iter reference)
import jax
import jax.numpy as jnp
from jax import lax
from jax.experimental import pallas as pl
from jax.experimental.pallas import tpu as pltpu

M_PER = 4096
N_PER = 1024
N_CHUNKS = 16
CH = M_PER // N_CHUNKS


def kernel(x):
    def body(x_hbm, out_ref, stage, conv, copy_sems, local_sems,
             send_sems, recv_sems):
        my_x = lax.axis_index("x")
        my_y = lax.axis_index("y")
        my_z = lax.axis_index("z")
        other_x = 1 - my_x
        partner = (other_x, my_y, my_z)

        def copy_in(c):
            return pltpu.make_async_copy(
                x_hbm.at[pl.ds(c * CH, CH), :],
                stage.at[c % 2],
                copy_sems.at[c % 2],
            )

        barrier_sem = pltpu.get_barrier_semaphore()
        pl.semaphore_signal(
            barrier_sem,
            inc=1,
            device_id=partner,
            device_id_type=pl.DeviceIdType.MESH,
        )
        copy_in(0).start()
        if N_CHUNKS > 1:
            copy_in(1).start()
        pl.semaphore_wait(barrier_sem, 1)

        rdmas = []
        locals_ = []
        for c in range(N_CHUNKS):
            slot = c % 2
            if c >= 2:
                rdmas[c - 2].wait_send()
                locals_[c - 2].wait()
            copy_in(c).wait()
            conv[slot, 0] = stage[slot][:, :N_PER].astype(jnp.bfloat16)
            conv[slot, 1] = stage[slot][:, N_PER:].astype(jnp.bfloat16)
            if c + 2 < N_CHUNKS:
                copy_in(c + 2).start()

            dst_rows = pl.ds(my_x * M_PER + c * CH, CH)
            rdma = pltpu.make_async_remote_copy(
                src_ref=conv.at[slot, other_x],
                dst_ref=out_ref.at[dst_rows, :],
                send_sem=send_sems.at[c],
                recv_sem=recv_sems.at[c],
                device_id=partner,
                device_id_type=pl.DeviceIdType.MESH,
            )
            rdma.start()
            rdmas.append(rdma)

            loc = pltpu.make_async_copy(
                conv.at[slot, my_x],
                out_ref.at[dst_rows, :],
                local_sems.at[slot],
            )
            loc.start()
            locals_.append(loc)

        for c in range(max(N_CHUNKS - 2, 0), N_CHUNKS):
            rdmas[c].wait_send()
            locals_[c].wait()
        for c in range(N_CHUNKS):
            rdmas[c].wait_recv()

    return pl.pallas_call(
        body,
        out_shape=jax.ShapeDtypeStruct((2 * M_PER, N_PER), jnp.bfloat16),
        in_specs=[pl.BlockSpec(memory_space=pl.MemorySpace.ANY)],
        out_specs=pl.BlockSpec(memory_space=pltpu.MemorySpace.HBM),
        scratch_shapes=[
            pltpu.VMEM((2, CH, 2 * N_PER), jnp.float32),
            pltpu.VMEM((2, 2, CH, N_PER), jnp.bfloat16),
            pltpu.SemaphoreType.DMA((2,)),
            pltpu.SemaphoreType.DMA((2,)),
            pltpu.SemaphoreType.DMA((N_CHUNKS,)),
            pltpu.SemaphoreType.DMA((N_CHUNKS,)),
        ],
        compiler_params=pltpu.CompilerParams(collective_id=0),
    )(x)


# device time: 100964 ns/iter; 1.0016x vs baseline; 1.0016x over previous
import jax
import jax.numpy as jnp
from jax import lax
from jax.experimental import pallas as pl
from jax.experimental.pallas import tpu as pltpu

M_PER = 4096
N_PER = 1024
N_CHUNKS = 32
CH = M_PER // N_CHUNKS


def kernel(x):
    def body(x_hbm, out_ref, stage, conv, copy_sems, local_sems,
             send_sems, recv_sems):
        my_x = lax.axis_index("x")
        my_y = lax.axis_index("y")
        my_z = lax.axis_index("z")
        other_x = 1 - my_x
        partner = (other_x, my_y, my_z)

        def copy_in(c):
            return pltpu.make_async_copy(
                x_hbm.at[pl.ds(c * CH, CH), :],
                stage.at[c % 2],
                copy_sems.at[c % 2],
            )

        barrier_sem = pltpu.get_barrier_semaphore()
        pl.semaphore_signal(
            barrier_sem,
            inc=1,
            device_id=partner,
            device_id_type=pl.DeviceIdType.MESH,
        )
        copy_in(0).start()
        if N_CHUNKS > 1:
            copy_in(1).start()
        pl.semaphore_wait(barrier_sem, 1)

        rdmas = []
        locals_ = []
        for c in range(N_CHUNKS):
            slot = c % 2
            if c >= 2:
                rdmas[c - 2].wait_send()
                locals_[c - 2].wait()
            copy_in(c).wait()
            conv[slot, 0] = stage[slot][:, :N_PER].astype(jnp.bfloat16)
            conv[slot, 1] = stage[slot][:, N_PER:].astype(jnp.bfloat16)
            if c + 2 < N_CHUNKS:
                copy_in(c + 2).start()

            dst_rows = pl.ds(my_x * M_PER + c * CH, CH)
            rdma = pltpu.make_async_remote_copy(
                src_ref=conv.at[slot, other_x],
                dst_ref=out_ref.at[dst_rows, :],
                send_sem=send_sems.at[c],
                recv_sem=recv_sems.at[c],
                device_id=partner,
                device_id_type=pl.DeviceIdType.MESH,
            )
            rdma.start()
            rdmas.append(rdma)

            loc = pltpu.make_async_copy(
                conv.at[slot, my_x],
                out_ref.at[dst_rows, :],
                local_sems.at[slot],
            )
            loc.start()
            locals_.append(loc)

        for c in range(max(N_CHUNKS - 2, 0), N_CHUNKS):
            rdmas[c].wait_send()
            locals_[c].wait()
        for c in range(N_CHUNKS):
            rdmas[c].wait_recv()

    return pl.pallas_call(
        body,
        out_shape=jax.ShapeDtypeStruct((2 * M_PER, N_PER), jnp.bfloat16),
        in_specs=[pl.BlockSpec(memory_space=pl.MemorySpace.ANY)],
        out_specs=pl.BlockSpec(memory_space=pltpu.MemorySpace.HBM),
        scratch_shapes=[
            pltpu.VMEM((2, CH, 2 * N_PER), jnp.float32),
            pltpu.VMEM((2, 2, CH, N_PER), jnp.bfloat16),
            pltpu.SemaphoreType.DMA((2,)),
            pltpu.SemaphoreType.DMA((2,)),
            pltpu.SemaphoreType.DMA((N_CHUNKS,)),
            pltpu.SemaphoreType.DMA((N_CHUNKS,)),
        ],
        compiler_params=pltpu.CompilerParams(collective_id=0),
    )(x)
